# baseline (speedup 1.0000x reference)
import jax
import jax.numpy as jnp
import numpy as np
from jax.experimental import pallas as pl
from jax.experimental.pallas import tpu as pltpu

_ORDER = 6
_VOCAB = 100000
_WIDTH = 256
_D_SCENE = 128
_BATCH = 32
_TILE = 4096
_CH = 512
_NCH = _TILE // _CH
_NT = (_VOCAB + _TILE - 1) // _TILE


def _rotl(x, r):
    return jax.lax.shift_left(x, jnp.uint32(r)) | jax.lax.shift_right_logical(
        x, jnp.uint32(32 - r)
    )


def _gumbel_chunk(ks, lane_u, id_ch):
    ks0, ks1, ks2 = ks
    x1 = lane_u + (ks1 + id_ch.astype(jnp.uint32) * jnp.uint32(_CH))
    x0 = x1 + ks0
    x1 = _rotl(x1, 13)
    x1 = x1 ^ x0
    rot = ((13, 15, 26, 6), (17, 29, 16, 24))
    for i in range(5):
        first = 1 if i == 0 else 0
        for r in rot[i % 2][first:]:
            x0 = x0 + x1
            x1 = _rotl(x1, r)
            x1 = x1 ^ x0
        x0 = x0 + ks[(i + 1) % 3]
        x1 = x1 + (ks[(i + 2) % 3] + jnp.uint32(i + 1))
    bits = x0 ^ x1
    fb = jax.lax.shift_right_logical(bits, jnp.uint32(9)) | jnp.uint32(0x3F800000)
    f = jax.lax.bitcast_convert_type(fb, jnp.float32) - jnp.float32(1.0)
    u = jnp.maximum(f, jnp.float32(np.finfo(np.float32).tiny))
    return -jnp.log(-jnp.log(u))


def _emb_row_copies(emb_ref, h0_ref, widx_sref, slot, sem):
    cps = []
    for b in range(_BATCH):
        row = widx_sref[b, 0] + 1
        cps.append(
            pltpu.make_async_copy(
                emb_ref.at[pl.ds(row, 1), :],
                h0_ref.at[pl.ds(b, 1), pl.ds(_D_SCENE + slot * _WIDTH, _WIDTH)],
                sem,
            )
        )
    return cps


def _kernel(
    scene_ref, emb_ref, win_ref, bin_ref, wh1_ref, bh1_ref, wout_ref,
    bout_ref, k1_ref, k2_ref, out_ref,
    h0_ref, h2_ref, acc_s_ref, acc_i_ref, e0_ref, wvec_ref, widx_sref,
    sem_e, sem_w,
):
    s = pl.program_id(0)
    t = pl.program_id(1)

    @pl.when((s == 0) & (t == 0))
    def _():
        cp = pltpu.make_async_copy(emb_ref.at[pl.ds(0, 1), :], e0_ref, sem_w)
        cp.start()
        h0_ref[:, 0:_D_SCENE] = scene_ref[...]
        cp.wait()
        e0b = jnp.broadcast_to(e0_ref[...], (_BATCH, _WIDTH))
        for i in range(_ORDER):
            h0_ref[:, _D_SCENE + i * _WIDTH:_D_SCENE + (i + 1) * _WIDTH] = e0b

    @pl.when((s > 0) & (t == 0))
    def _():
        for cp in _emb_row_copies(emb_ref, h0_ref, widx_sref, s - 1, sem_e):
            cp.wait()

    @pl.when(t == 0)
    def _():
        h0 = h0_ref[...]
        h1 = jnp.maximum(
            jnp.dot(h0, win_ref[...], preferred_element_type=jnp.float32)
            + bin_ref[...],
            0.0,
        )
        h2 = jnp.maximum(
            jnp.dot(h1, wh1_ref[...], preferred_element_type=jnp.float32)
            + bh1_ref[...],
            0.0,
        )
        h2_ref[...] = h2
        acc_s_ref[...] = jnp.full((_BATCH, _CH), -jnp.inf, jnp.float32)
        acc_i_ref[...] = jnp.zeros((_BATCH, _CH), jnp.int32)

    k1 = k1_ref[0]
    k2 = k2_ref[0]
    ks = (k1, k2, k1 ^ k2 ^ jnp.uint32(0x1BD11BDA))
    lane = jax.lax.broadcasted_iota(jnp.int32, (_BATCH, _CH), 1)
    lane_u = lane.astype(jnp.uint32)
    h2 = h2_ref[...]

    acc_s = acc_s_ref[...]
    acc_i = acc_i_ref[...]

    def dot_chunk(c):
        return (
            jnp.dot(
                h2,
                wout_ref[:, c * _CH:(c + 1) * _CH],
                preferred_element_type=jnp.float32,
            )
            + bout_ref[:, c * _CH:(c + 1) * _CH]
        )

    def consume(z, g, id_ch, acc_s, acc_i):
        flows = jnp.maximum(z, 0.0) + jnp.log1p(jnp.exp(-jnp.abs(z)))
        score = jnp.log(flows + 1e-9) + g
        score = jnp.where(lane < _VOCAB - id_ch * _CH, score, -jnp.inf)
        upd = score > acc_s
        return (
            jnp.where(upd, score, acc_s),
            jnp.where(upd, id_ch, acc_i),
        )

    for c in range(_NCH):
        g_c = _gumbel_chunk(ks, lane_u, t * _NCH + c)
        z_c = dot_chunk(c)
        acc_s, acc_i = consume(z_c, g_c, t * _NCH + c, acc_s, acc_i)
    acc_s_ref[...] = acc_s
    acc_i_ref[...] = acc_i

    @pl.when(t == _NT - 1)
    def _():
        m = jnp.max(acc_s, axis=1, keepdims=True)
        gidx = acc_i * _CH + lane
        cand = jnp.where(acc_s == m, gidx, jnp.int32(2**30))
        winner = jnp.min(cand, axis=1, keepdims=True)
        out_ref[0] = winner
        wvec_ref[...] = winner
        cpw = pltpu.make_async_copy(wvec_ref, widx_sref, sem_w)
        cpw.start()
        cpw.wait()

        @pl.when(s < _ORDER - 1)
        def _():
            for cp in _emb_row_copies(emb_ref, h0_ref, widx_sref, s, sem_e):
                cp.start()


def _run(scene, emb, W_in, b_in2, Wh1, bh1_2, W_out, b_out2, k1s, k2s):
    d_in = _D_SCENE + _ORDER * _WIDTH
    return pl.pallas_call(
        _kernel,
        grid=(_ORDER, _NT),
        in_specs=[
            pl.BlockSpec((_BATCH, _D_SCENE), lambda s, t: (0, 0)),
            pl.BlockSpec(memory_space=pl.ANY),
            pl.BlockSpec((d_in, _WIDTH), lambda s, t: (0, 0)),
            pl.BlockSpec((1, _WIDTH), lambda s, t: (0, 0)),
            pl.BlockSpec((_WIDTH, _WIDTH), lambda s, t: (0, 0)),
            pl.BlockSpec((1, _WIDTH), lambda s, t: (0, 0)),
            pl.BlockSpec((_WIDTH, _TILE), lambda s, t: (0, t)),
            pl.BlockSpec((1, _TILE), lambda s, t: (0, t)),
            pl.BlockSpec((1, _BATCH, 1), lambda s, t: (s, 0, 0)),
            pl.BlockSpec((1, _BATCH, 1), lambda s, t: (s, 0, 0)),
        ],
        out_specs=pl.BlockSpec((1, _BATCH, 1), lambda s, t: (s, 0, 0)),
        out_shape=jax.ShapeDtypeStruct((_ORDER, _BATCH, 1), jnp.int32),
        scratch_shapes=[
            pltpu.VMEM((_BATCH, _D_SCENE + _ORDER * _WIDTH), jnp.float32),
            pltpu.VMEM((_BATCH, _WIDTH), jnp.float32),
            pltpu.VMEM((_BATCH, _CH), jnp.float32),
            pltpu.VMEM((_BATCH, _CH), jnp.int32),
            pltpu.VMEM((1, _WIDTH), jnp.float32),
            pltpu.VMEM((_BATCH, 1), jnp.int32),
            pltpu.SMEM((_BATCH, 1), jnp.int32),
            pltpu.SemaphoreType.DMA,
            pltpu.SemaphoreType.DMA,
        ],
    )(scene, emb, W_in, b_in2, Wh1, bh1_2, W_out, b_out2, k1s, k2s)


def kernel(scene, emb, W_in, b_in, Wh1, bh1, W_out, b_out, seed):
    base = jax.random.key(seed)
    keys = jax.random.split(base, _BATCH)

    def derive(k):
        _, key = jax.random.split(k)
        step_keys = jax.random.split(key, _ORDER)
        eak = jax.vmap(lambda sk: jax.random.split(sk))(step_keys)
        return eak[:, 1]

    action_keys = jax.vmap(derive)(keys)
    kd = jax.random.key_data(action_keys).astype(jnp.uint32)

    paths = _run(
        scene, emb, W_in, b_in.reshape(1, _WIDTH), Wh1,
        bh1.reshape(1, _WIDTH), W_out, b_out.reshape(1, _VOCAB),
        kd[:, :, 0].T.reshape(_ORDER, _BATCH, 1),
        kd[:, :, 1].T.reshape(_ORDER, _BATCH, 1),
    )
    return paths.reshape(_ORDER, _BATCH).T


# speedup vs baseline: 1.0182x; 1.0182x over previous
import jax
import jax.numpy as jnp
import numpy as np
from jax.experimental import pallas as pl
from jax.experimental.pallas import tpu as pltpu

_ORDER = 6
_VOCAB = 100000
_WIDTH = 256
_D_SCENE = 128
_BATCH = 32
_TILE = 4096
_CH = 256
_NCH = _TILE // _CH
_NT = (_VOCAB + _TILE - 1) // _TILE


def _rotl(x, r):
    return jax.lax.shift_left(x, jnp.uint32(r)) | jax.lax.shift_right_logical(
        x, jnp.uint32(32 - r)
    )


def _gumbel_chunk(ks, lane_u, id_ch):
    ks0, ks1, ks2 = ks
    x1 = lane_u + (ks1 + id_ch.astype(jnp.uint32) * jnp.uint32(_CH))
    x0 = x1 + ks0
    x1 = _rotl(x1, 13)
    x1 = x1 ^ x0
    rot = ((13, 15, 26, 6), (17, 29, 16, 24))
    for i in range(5):
        first = 1 if i == 0 else 0
        for r in rot[i % 2][first:]:
            x0 = x0 + x1
            x1 = _rotl(x1, r)
            x1 = x1 ^ x0
        x0 = x0 + ks[(i + 1) % 3]
        x1 = x1 + (ks[(i + 2) % 3] + jnp.uint32(i + 1))
    bits = x0 ^ x1
    fb = jax.lax.shift_right_logical(bits, jnp.uint32(9)) | jnp.uint32(0x3F800000)
    f = jax.lax.bitcast_convert_type(fb, jnp.float32) - jnp.float32(1.0)
    u = jnp.maximum(f, jnp.float32(np.finfo(np.float32).tiny))
    return -jnp.log(-jnp.log(u))


def _emb_row_copies(emb_ref, h0_ref, widx_sref, slot, sem):
    cps = []
    for b in range(_BATCH):
        row = widx_sref[b, 0] + 1
        cps.append(
            pltpu.make_async_copy(
                emb_ref.at[pl.ds(row, 1), :],
                h0_ref.at[pl.ds(b, 1), pl.ds(_D_SCENE + slot * _WIDTH, _WIDTH)],
                sem,
            )
        )
    return cps


def _kernel(
    scene_ref, emb_ref, win_ref, bin_ref, wh1_ref, bh1_ref, wout_ref,
    bout_ref, k1_ref, k2_ref, out_ref,
    h0_ref, h2_ref, acc_s_ref, acc_i_ref, e0_ref, wvec_ref, widx_sref,
    sem_e, sem_w,
):
    s = pl.program_id(0)
    t = pl.program_id(1)

    @pl.when((s == 0) & (t == 0))
    def _():
        cp = pltpu.make_async_copy(emb_ref.at[pl.ds(0, 1), :], e0_ref, sem_w)
        cp.start()
        h0_ref[:, 0:_D_SCENE] = scene_ref[...]
        cp.wait()
        e0b = jnp.broadcast_to(e0_ref[...], (_BATCH, _WIDTH))
        for i in range(_ORDER):
            h0_ref[:, _D_SCENE + i * _WIDTH:_D_SCENE + (i + 1) * _WIDTH] = e0b

    @pl.when((s > 0) & (t == 0))
    def _():
        for cp in _emb_row_copies(emb_ref, h0_ref, widx_sref, s - 1, sem_e):
            cp.wait()

    @pl.when(t == 0)
    def _():
        h0 = h0_ref[...]
        h1 = jnp.maximum(
            jnp.dot(h0, win_ref[...], preferred_element_type=jnp.float32)
            + bin_ref[...],
            0.0,
        )
        h2 = jnp.maximum(
            jnp.dot(h1, wh1_ref[...], preferred_element_type=jnp.float32)
            + bh1_ref[...],
            0.0,
        )
        h2_ref[...] = h2
        acc_s_ref[...] = jnp.full((_BATCH, _CH), -jnp.inf, jnp.float32)
        acc_i_ref[...] = jnp.zeros((_BATCH, _CH), jnp.int32)

    k1 = k1_ref[0]
    k2 = k2_ref[0]
    ks = (k1, k2, k1 ^ k2 ^ jnp.uint32(0x1BD11BDA))
    lane = jax.lax.broadcasted_iota(jnp.int32, (_BATCH, _CH), 1)
    lane_u = lane.astype(jnp.uint32)

    def dot_chunk(c):
        return (
            jnp.dot(
                h2_ref[...],
                wout_ref[:, c * _CH:(c + 1) * _CH],
                preferred_element_type=jnp.float32,
            )
            + bout_ref[:, c * _CH:(c + 1) * _CH]
        )

    def consume(z, g, id_ch):
        flows = jnp.maximum(z, 0.0) + jnp.log1p(jnp.exp(-jnp.abs(z)))
        score = jnp.log(flows + 1e-9) + g
        score = jnp.where(lane < _VOCAB - id_ch * _CH, score, -jnp.inf)
        acc_s = acc_s_ref[...]
        upd = score > acc_s
        acc_s_ref[...] = jnp.where(upd, score, acc_s)
        acc_i_ref[...] = jnp.where(upd, id_ch, acc_i_ref[...])

    for c in range(_NCH):
        g_c = _gumbel_chunk(ks, lane_u, t * _NCH + c)
        z_c = dot_chunk(c)
        consume(z_c, g_c, t * _NCH + c)

    @pl.when(t == _NT - 1)
    def _():
        acc_s = acc_s_ref[...]
        acc_i = acc_i_ref[...]
        m = jnp.max(acc_s, axis=1, keepdims=True)
        gidx = acc_i * _CH + lane
        cand = jnp.where(acc_s == m, gidx, jnp.int32(2**30))
        winner = jnp.min(cand, axis=1, keepdims=True)
        out_ref[0] = winner
        wvec_ref[...] = winner
        cpw = pltpu.make_async_copy(wvec_ref, widx_sref, sem_w)
        cpw.start()
        cpw.wait()

        @pl.when(s < _ORDER - 1)
        def _():
            for cp in _emb_row_copies(emb_ref, h0_ref, widx_sref, s, sem_e):
                cp.start()


def _run(scene, emb, W_in, b_in2, Wh1, bh1_2, W_out, b_out2, k1s, k2s):
    d_in = _D_SCENE + _ORDER * _WIDTH
    return pl.pallas_call(
        _kernel,
        grid=(_ORDER, _NT),
        in_specs=[
            pl.BlockSpec((_BATCH, _D_SCENE), lambda s, t: (0, 0)),
            pl.BlockSpec(memory_space=pl.ANY),
            pl.BlockSpec((d_in, _WIDTH), lambda s, t: (0, 0)),
            pl.BlockSpec((1, _WIDTH), lambda s, t: (0, 0)),
            pl.BlockSpec((_WIDTH, _WIDTH), lambda s, t: (0, 0)),
            pl.BlockSpec((1, _WIDTH), lambda s, t: (0, 0)),
            pl.BlockSpec((_WIDTH, _TILE), lambda s, t: (0, t)),
            pl.BlockSpec((1, _TILE), lambda s, t: (0, t)),
            pl.BlockSpec((1, _BATCH, 1), lambda s, t: (s, 0, 0)),
            pl.BlockSpec((1, _BATCH, 1), lambda s, t: (s, 0, 0)),
        ],
        out_specs=pl.BlockSpec((1, _BATCH, 1), lambda s, t: (s, 0, 0)),
        out_shape=jax.ShapeDtypeStruct((_ORDER, _BATCH, 1), jnp.int32),
        scratch_shapes=[
            pltpu.VMEM((_BATCH, _D_SCENE + _ORDER * _WIDTH), jnp.float32),
            pltpu.VMEM((_BATCH, _WIDTH), jnp.float32),
            pltpu.VMEM((_BATCH, _CH), jnp.float32),
            pltpu.VMEM((_BATCH, _CH), jnp.int32),
            pltpu.VMEM((1, _WIDTH), jnp.float32),
            pltpu.VMEM((_BATCH, 1), jnp.int32),
            pltpu.SMEM((_BATCH, 1), jnp.int32),
            pltpu.SemaphoreType.DMA,
            pltpu.SemaphoreType.DMA,
        ],
    )(scene, emb, W_in, b_in2, Wh1, bh1_2, W_out, b_out2, k1s, k2s)


def kernel(scene, emb, W_in, b_in, Wh1, bh1, W_out, b_out, seed):
    base = jax.random.key(seed)
    keys = jax.random.split(base, _BATCH)

    def derive(k):
        _, key = jax.random.split(k)
        step_keys = jax.random.split(key, _ORDER)
        eak = jax.vmap(lambda sk: jax.random.split(sk))(step_keys)
        return eak[:, 1]

    action_keys = jax.vmap(derive)(keys)
    kd = jax.random.key_data(action_keys).astype(jnp.uint32)

    paths = _run(
        scene, emb, W_in, b_in.reshape(1, _WIDTH), Wh1,
        bh1.reshape(1, _WIDTH), W_out, b_out.reshape(1, _VOCAB),
        kd[:, :, 0].T.reshape(_ORDER, _BATCH, 1),
        kd[:, :, 1].T.reshape(_ORDER, _BATCH, 1),
    )
    return paths.reshape(_ORDER, _BATCH).T


# speedup vs baseline: 1.0193x; 1.0011x over previous
import jax
import jax.numpy as jnp
import numpy as np
from jax.experimental import pallas as pl
from jax.experimental.pallas import tpu as pltpu

_ORDER = 6
_VOCAB = 100000
_WIDTH = 256
_D_SCENE = 128
_BATCH = 32
_TILE = 4096
_CH = 256
_NCH = _TILE // _CH
_NT = (_VOCAB + _TILE - 1) // _TILE


def _rotl(x, r):
    return jax.lax.shift_left(x, jnp.uint32(r)) | jax.lax.shift_right_logical(
        x, jnp.uint32(32 - r)
    )


def _key_schedule(k1, k2):
    ks = (k1, k2, k1 ^ k2 ^ jnp.uint32(0x1BD11BDA))
    inj0 = tuple(ks[(i + 1) % 3] for i in range(5))
    inj1 = tuple(ks[(i + 2) % 3] + jnp.uint32(i + 1) for i in range(5))
    return ks, inj0, inj1


def _gumbel_chunk(ks, inj0, inj1, lane_u, id_ch):
    ks0, ks1, _ = ks
    x1 = lane_u + (ks1 + id_ch.astype(jnp.uint32) * jnp.uint32(_CH))
    x0 = x1 + ks0
    x1 = _rotl(x1, 13)
    x1 = x1 ^ x0
    rot = ((13, 15, 26, 6), (17, 29, 16, 24))
    for i in range(5):
        first = 1 if i == 0 else 0
        for r in rot[i % 2][first:]:
            x0 = x0 + x1
            x1 = _rotl(x1, r)
            x1 = x1 ^ x0
        x0 = x0 + inj0[i]
        x1 = x1 + inj1[i]
    bits = x0 ^ x1
    fb = jax.lax.shift_right_logical(bits, jnp.uint32(9)) | jnp.uint32(0x3F800000)
    f = jax.lax.bitcast_convert_type(fb, jnp.float32) - jnp.float32(1.0)
    u = jnp.maximum(f, jnp.float32(np.finfo(np.float32).tiny))
    return -jnp.log(-jnp.log(u))


def _emb_row_copies(emb_ref, h0_ref, widx_sref, slot, sem):
    cps = []
    for b in range(_BATCH):
        row = widx_sref[b, 0] + 1
        cps.append(
            pltpu.make_async_copy(
                emb_ref.at[pl.ds(row, 1), :],
                h0_ref.at[pl.ds(b, 1), pl.ds(_D_SCENE + slot * _WIDTH, _WIDTH)],
                sem,
            )
        )
    return cps


def _kernel(
    scene_ref, emb_ref, win_ref, bin_ref, wh1_ref, bh1_ref, wout_ref,
    bout_ref, k1_ref, k2_ref, out_ref,
    h0_ref, h2_ref, acc_s_ref, acc_i_ref, e0_ref, wvec_ref, widx_sref,
    sem_e, sem_w,
):
    s = pl.program_id(0)
    t = pl.program_id(1)

    @pl.when((s == 0) & (t == 0))
    def _():
        cp = pltpu.make_async_copy(emb_ref.at[pl.ds(0, 1), :], e0_ref, sem_w)
        cp.start()
        h0_ref[:, 0:_D_SCENE] = scene_ref[...]
        cp.wait()
        e0b = jnp.broadcast_to(e0_ref[...], (_BATCH, _WIDTH))
        for i in range(_ORDER):
            h0_ref[:, _D_SCENE + i * _WIDTH:_D_SCENE + (i + 1) * _WIDTH] = e0b

    @pl.when((s > 0) & (t == 0))
    def _():
        for cp in _emb_row_copies(emb_ref, h0_ref, widx_sref, s - 1, sem_e):
            cp.wait()

    @pl.when(t == 0)
    def _():
        h0 = h0_ref[...]
        h1 = jnp.maximum(
            jnp.dot(h0, win_ref[...], preferred_element_type=jnp.float32)
            + bin_ref[...],
            0.0,
        )
        h2 = jnp.maximum(
            jnp.dot(h1, wh1_ref[...], preferred_element_type=jnp.float32)
            + bh1_ref[...],
            0.0,
        )
        h2_ref[...] = h2
        acc_s_ref[...] = jnp.full((_BATCH, _CH), -jnp.inf, jnp.float32)
        acc_i_ref[...] = jnp.zeros((_BATCH, _CH), jnp.int32)

    ks, inj0, inj1 = _key_schedule(k1_ref[0], k2_ref[0])
    lane = jax.lax.broadcasted_iota(jnp.int32, (_BATCH, _CH), 1)
    lane_u = lane.astype(jnp.uint32)

    def dot_chunk(c):
        return (
            jnp.dot(
                h2_ref[...],
                wout_ref[:, c * _CH:(c + 1) * _CH],
                preferred_element_type=jnp.float32,
            )
            + bout_ref[:, c * _CH:(c + 1) * _CH]
        )

    def consume(z, g, id_ch):
        flows = jnp.maximum(z, 0.0) + jnp.log1p(jnp.exp(-jnp.abs(z)))
        score = jnp.log(flows + 1e-9) + g
        score = jnp.where(lane < _VOCAB - id_ch * _CH, score, -jnp.inf)
        acc_s = acc_s_ref[...]
        upd = score > acc_s
        acc_s_ref[...] = jnp.where(upd, score, acc_s)
        acc_i_ref[...] = jnp.where(upd, id_ch, acc_i_ref[...])

    g_prev = _gumbel_chunk(ks, inj0, inj1, lane_u, t * _NCH)
    z_prev = dot_chunk(0)
    for c in range(1, _NCH):
        g_c = _gumbel_chunk(ks, inj0, inj1, lane_u, t * _NCH + c)
        z_c = dot_chunk(c)
        consume(z_prev, g_prev, t * _NCH + (c - 1))
        g_prev, z_prev = g_c, z_c
    consume(z_prev, g_prev, t * _NCH + (_NCH - 1))

    @pl.when(t == _NT - 1)
    def _():
        acc_s = acc_s_ref[...]
        acc_i = acc_i_ref[...]
        m = jnp.max(acc_s, axis=1, keepdims=True)
        gidx = acc_i * _CH + lane
        cand = jnp.where(acc_s == m, gidx, jnp.int32(2**30))
        winner = jnp.min(cand, axis=1, keepdims=True)
        out_ref[0] = winner
        wvec_ref[...] = winner
        cpw = pltpu.make_async_copy(wvec_ref, widx_sref, sem_w)
        cpw.start()
        cpw.wait()

        @pl.when(s < _ORDER - 1)
        def _():
            for cp in _emb_row_copies(emb_ref, h0_ref, widx_sref, s, sem_e):
                cp.start()


def _run(scene, emb, W_in, b_in2, Wh1, bh1_2, W_out, b_out2, k1s, k2s):
    d_in = _D_SCENE + _ORDER * _WIDTH
    return pl.pallas_call(
        _kernel,
        grid=(_ORDER, _NT),
        in_specs=[
            pl.BlockSpec((_BATCH, _D_SCENE), lambda s, t: (0, 0)),
            pl.BlockSpec(memory_space=pl.ANY),
            pl.BlockSpec((d_in, _WIDTH), lambda s, t: (0, 0)),
            pl.BlockSpec((1, _WIDTH), lambda s, t: (0, 0)),
            pl.BlockSpec((_WIDTH, _WIDTH), lambda s, t: (0, 0)),
            pl.BlockSpec((1, _WIDTH), lambda s, t: (0, 0)),
            pl.BlockSpec((_WIDTH, _TILE), lambda s, t: (0, t)),
            pl.BlockSpec((1, _TILE), lambda s, t: (0, t)),
            pl.BlockSpec((1, _BATCH, 1), lambda s, t: (s, 0, 0)),
            pl.BlockSpec((1, _BATCH, 1), lambda s, t: (s, 0, 0)),
        ],
        out_specs=pl.BlockSpec((1, _BATCH, 1), lambda s, t: (s, 0, 0)),
        out_shape=jax.ShapeDtypeStruct((_ORDER, _BATCH, 1), jnp.int32),
        scratch_shapes=[
            pltpu.VMEM((_BATCH, _D_SCENE + _ORDER * _WIDTH), jnp.float32),
            pltpu.VMEM((_BATCH, _WIDTH), jnp.float32),
            pltpu.VMEM((_BATCH, _CH), jnp.float32),
            pltpu.VMEM((_BATCH, _CH), jnp.int32),
            pltpu.VMEM((1, _WIDTH), jnp.float32),
            pltpu.VMEM((_BATCH, 1), jnp.int32),
            pltpu.SMEM((_BATCH, 1), jnp.int32),
            pltpu.SemaphoreType.DMA,
            pltpu.SemaphoreType.DMA,
        ],
    )(scene, emb, W_in, b_in2, Wh1, bh1_2, W_out, b_out2, k1s, k2s)


def kernel(scene, emb, W_in, b_in, Wh1, bh1, W_out, b_out, seed):
    base = jax.random.key(seed)
    keys = jax.random.split(base, _BATCH)

    def derive(k):
        _, key = jax.random.split(k)
        step_keys = jax.random.split(key, _ORDER)
        eak = jax.vmap(lambda sk: jax.random.split(sk))(step_keys)
        return eak[:, 1]

    action_keys = jax.vmap(derive)(keys)
    kd = jax.random.key_data(action_keys).astype(jnp.uint32)

    paths = _run(
        scene, emb, W_in, b_in.reshape(1, _WIDTH), Wh1,
        bh1.reshape(1, _WIDTH), W_out, b_out.reshape(1, _VOCAB),
        kd[:, :, 0].T.reshape(_ORDER, _BATCH, 1),
        kd[:, :, 1].T.reshape(_ORDER, _BATCH, 1),
    )
    return paths.reshape(_ORDER, _BATCH).T
